# Initial kernel scaffold; baseline (speedup 1.0000x reference)
import jax
import jax.numpy as jnp
from jax.experimental import pallas as pl


def kernel(feature, index, consisenty, W_in, b_in, rel_weight, root_w, rgcn_bias, W_out, b_out):
    raise NotImplementedError("write your pallas kernel here")


# speedup vs baseline: 10.5196x; 10.5196x over previous
import jax
import jax.numpy as jnp
from jax import lax
from jax.experimental import pallas as pl
from jax.experimental.pallas import tpu as pltpu
from jax.experimental.pallas import tpu_sc as plsc

N = 10000
NP = 10240
E = 320000
DH = 128
HH = 64
NR = 2
NC = 2
NS = 16
EPT = E // NS
EPC = E // (NC * NS)
BLK = 2000
CHUNK = 80
NCH = BLK // CHUNK
RPT = 2 * NP // NS
ZR = 256

_SC_PARAMS = pltpu.CompilerParams(use_tc_tiling_on_sc=False)


def _mesh():
    return plsc.VectorSubcoreMesh(core_axis_name="c", subcore_axis_name="s",
                                  num_cores=NC, num_subcores=NS)


def _make_sc_agg():
    scratch = [
        pltpu.VMEM_SHARED((2 * NP, HH), jnp.float32),
        pltpu.VMEM((BLK,), jnp.int32),
        pltpu.VMEM((BLK,), jnp.int32),
        pltpu.VMEM((BLK,), jnp.int32),
        pltpu.VMEM((NCH, CHUNK), jnp.int32),
        pltpu.VMEM((NCH, CHUNK), jnp.int32),
        pltpu.VMEM((CHUNK, HH), jnp.float32),
        pltpu.VMEM((CHUNK, HH), jnp.float32),
        pltpu.VMEM((ZR, HH), jnp.float32),
        pltpu.SemaphoreType.DMA,
        pltpu.SemaphoreType.DMA,
    ]

    def body(tbl, src, dst, ety, agg_out, acc_sh, srcb, dstb, etyb,
             gidx, sidx, g0, g1, zbuf, sem0, sem1):
        c = lax.axis_index("c")
        s = lax.axis_index("s")
        zv = jnp.zeros((16,), jnp.float32)

        def zrow(i, _):
            for k in range(HH // 16):
                zbuf[i, pl.ds(k * 16, 16)] = zv
            return 0
        lax.fori_loop(0, ZR, zrow, 0)

        rbase = s * RPT

        def zacc(b, _):
            pltpu.sync_copy(zbuf, acc_sh.at[pl.ds(rbase + b * ZR, ZR)])
            return 0
        lax.fori_loop(0, RPT // ZR, zacc, 0)

        plsc.subcore_barrier()

        ebase = s * EPT
        coff = c * N

        def blk_body(b, _):
            eb = ebase + b * BLK
            pltpu.sync_copy(src.at[pl.ds(eb, BLK)], srcb)
            pltpu.sync_copy(dst.at[pl.ds(eb, BLK)], dstb)
            pltpu.sync_copy(ety.at[pl.ds(eb, BLK)], etyb)

            def idx_body(j, _):
                for v in range(CHUNK // 16):
                    sl = pl.ds(j * CHUNK + v * 16, 16)
                    vd = pl.ds(v * 16, 16)
                    gidx[j, vd] = srcb[sl] + coff
                    sidx[j, vd] = etyb[sl] * NP + dstb[sl]
                return 0
            lax.fori_loop(0, NCH, idx_body, 0)

            bufs = (g0, g1)
            sems = (sem0, sem1)
            descs = {0: pltpu.async_copy(tbl.at[gidx.at[0]], g0, sem0)}
            for j in range(NCH):
                if j + 1 < NCH:
                    descs[j + 1] = pltpu.async_copy(
                        tbl.at[gidx.at[j + 1]], bufs[(j + 1) % 2],
                        sems[(j + 1) % 2])
                descs[j].wait()
                pltpu.sync_copy(bufs[j % 2], acc_sh.at[sidx.at[j]], add=True)
            return 0
        lax.fori_loop(0, EPT // BLK, blk_body, 0)

        plsc.subcore_barrier()

        pltpu.sync_copy(acc_sh.at[pl.ds(rbase, RPT)],
                        agg_out.at[c].at[pl.ds(rbase, RPT)])

    return pl.kernel(
        body, out_type=jax.ShapeDtypeStruct((NC, 2 * NP, HH), jnp.float32),
        mesh=_mesh(), scratch_types=scratch, compiler_params=_SC_PARAMS)


def _make_sc_counts():
    scratch = [
        pltpu.VMEM_SHARED((2 * NP, 16), jnp.float32),
        pltpu.VMEM((BLK,), jnp.int32),
        pltpu.VMEM((BLK,), jnp.int32),
        pltpu.VMEM((NCH, CHUNK), jnp.int32),
        pltpu.VMEM((CHUNK, 16), jnp.float32),
        pltpu.VMEM((ZR, 16), jnp.float32),
    ]

    def body(dst, ety, cnt_out, cnt_sh, dstb, etyb, sidx, ones, zcnt):
        c = lax.axis_index("c")
        s = lax.axis_index("s")
        zv = jnp.zeros((16,), jnp.float32)
        ov = jnp.ones((16,), jnp.float32)

        def zc(i, _):
            zcnt[i, pl.ds(0, 16)] = zv
            return 0
        lax.fori_loop(0, ZR, zc, 0)

        def oc(i, _):
            ones[i, pl.ds(0, 16)] = ov
            return 0
        lax.fori_loop(0, CHUNK, oc, 0)

        rbase = s * RPT

        def zcs(b, _):
            pltpu.sync_copy(zcnt, cnt_sh.at[pl.ds(rbase + b * ZR, ZR)])
            return 0
        lax.fori_loop(0, RPT // ZR, zcs, 0)

        plsc.subcore_barrier()

        ebase = (c * NS + s) * EPC

        def blk_body(b, _):
            eb = ebase + b * BLK
            pltpu.sync_copy(dst.at[pl.ds(eb, BLK)], dstb)
            pltpu.sync_copy(ety.at[pl.ds(eb, BLK)], etyb)

            def idx_body(j, _):
                for v in range(CHUNK // 16):
                    sl = pl.ds(j * CHUNK + v * 16, 16)
                    sidx[j, pl.ds(v * 16, 16)] = etyb[sl] * NP + dstb[sl]
                return 0
            lax.fori_loop(0, NCH, idx_body, 0)

            for j in range(NCH):
                pltpu.sync_copy(ones, cnt_sh.at[sidx.at[j]], add=True)
            return 0
        lax.fori_loop(0, EPC // BLK, blk_body, 0)

        plsc.subcore_barrier()

        pltpu.sync_copy(cnt_sh.at[pl.ds(rbase, RPT)],
                        cnt_out.at[c].at[pl.ds(rbase, RPT)])

    return pl.kernel(
        body, out_type=jax.ShapeDtypeStruct((NC, 2 * NP, 16), jnp.float32),
        mesh=_mesh(), scratch_types=scratch, compiler_params=_SC_PARAMS)


def _tc_input_proj(feature, W_in, b_in2):
    B = 1000

    def body(f_ref, w_ref, b_ref, x_ref, xh_ref):
        act = jnp.dot(f_ref[...], w_ref[...],
                      preferred_element_type=jnp.float32) + b_ref[...]
        act = jnp.where(act >= 0, act, 0.01 * act)
        x_ref[...] = act
        xh_ref[0] = act[:, :HH]
        xh_ref[1] = act[:, HH:]

    return pl.pallas_call(
        body,
        grid=(N // B,),
        in_specs=[
            pl.BlockSpec((B, 16), lambda i: (i, 0)),
            pl.BlockSpec((16, DH), lambda i: (0, 0)),
            pl.BlockSpec((1, DH), lambda i: (0, 0)),
        ],
        out_specs=[
            pl.BlockSpec((B, DH), lambda i: (i, 0)),
            pl.BlockSpec((2, B, HH), lambda i: (0, i, 0)),
        ],
        out_shape=[
            jax.ShapeDtypeStruct((N, DH), jnp.float32),
            jax.ShapeDtypeStruct((2, N, HH), jnp.float32),
        ],
    )(feature, W_in, b_in2)


def _tc_combine(x, agg4, cnt4, rel_w, root_w, bias2, final, W_out=None,
                b_out2=None):
    B = 1000

    def body(x_ref, a_ref, c_ref, rw_ref, rt_ref, b_ref, *refs):
        if final:
            wo_ref, bo_ref = refs[0], refs[1]
            refs = refs[2:]
        acc = jnp.dot(x_ref[...], rt_ref[...],
                      preferred_element_type=jnp.float32) + b_ref[...]
        for r in range(NR):
            cnt = c_ref[0, r, :, 0:1] + c_ref[1, r, :, 0:1]
            inv = 1.0 / jnp.maximum(cnt, 1.0)
            for h in range(2):
                m = a_ref[h, r] * inv
                acc += jnp.dot(m, rw_ref[r, h * HH:(h + 1) * HH, :],
                               preferred_element_type=jnp.float32)
        if final:
            o = jnp.dot(acc, wo_ref[...],
                        preferred_element_type=jnp.float32) + bo_ref[...]
            refs[0][...] = o
        else:
            refs[0][...] = acc
            refs[1][0] = acc[:, :HH]
            refs[1][1] = acc[:, HH:]

    in_specs = [
        pl.BlockSpec((B, DH), lambda i: (i, 0)),
        pl.BlockSpec((NC, NR, B, HH), lambda i: (0, 0, i, 0)),
        pl.BlockSpec((NC, NR, B, 16), lambda i: (0, 0, i, 0)),
        pl.BlockSpec((NR, DH, DH), lambda i: (0, 0, 0)),
        pl.BlockSpec((DH, DH), lambda i: (0, 0)),
        pl.BlockSpec((1, DH), lambda i: (0, 0)),
    ]
    args = [x, agg4, cnt4, rel_w, root_w, bias2]
    if final:
        in_specs += [
            pl.BlockSpec((DH, 3), lambda i: (0, 0)),
            pl.BlockSpec((1, 3), lambda i: (0, 0)),
        ]
        args += [W_out, b_out2]
        out_specs = pl.BlockSpec((B, 3), lambda i: (i, 0))
        out_shape = jax.ShapeDtypeStruct((N, 3), jnp.float32)
    else:
        out_specs = [
            pl.BlockSpec((B, DH), lambda i: (i, 0)),
            pl.BlockSpec((2, B, HH), lambda i: (0, i, 0)),
        ]
        out_shape = [
            jax.ShapeDtypeStruct((N, DH), jnp.float32),
            jax.ShapeDtypeStruct((2, N, HH), jnp.float32),
        ]

    return pl.pallas_call(body, grid=(N // B,), in_specs=in_specs,
                          out_specs=out_specs, out_shape=out_shape)(*args)


def kernel(feature, index, consisenty, W_in, b_in, rel_weight, root_w,
           rgcn_bias, W_out, b_out):
    src = index[0].astype(jnp.int32)
    dst = index[1].astype(jnp.int32)
    ety = consisenty.astype(jnp.int32)

    x0, xh0 = _tc_input_proj(feature.astype(jnp.float32), W_in,
                             b_in.reshape(1, DH))

    sc_agg = _make_sc_agg()
    sc_counts = _make_sc_counts()

    cnt = sc_counts(dst, ety)
    cnt4 = cnt.reshape(NC, NR, NP, 16)

    agg1 = sc_agg(xh0.reshape(2 * N, HH), src, dst, ety)

    x1, xh1 = _tc_combine(x0, agg1.reshape(NC, NR, NP, HH), cnt4,
                          rel_weight, root_w, rgcn_bias.reshape(1, DH),
                          final=False)

    agg2 = sc_agg(xh1.reshape(2 * N, HH), src, dst, ety)

    out = _tc_combine(x1, agg2.reshape(NC, NR, NP, HH), cnt4,
                      rel_weight, root_w, rgcn_bias.reshape(1, DH),
                      final=True, W_out=W_out, b_out2=b_out.reshape(1, 3))
    return out


# speedup vs baseline: 13.1503x; 1.2501x over previous
import jax
import jax.numpy as jnp
from jax import lax
from jax.experimental import pallas as pl
from jax.experimental.pallas import tpu as pltpu
from jax.experimental.pallas import tpu_sc as plsc

N = 10000
NP = 10240
E = 320000
DH = 128
HH = 64
NR = 2
NC = 2
NS = 16
EPT = E // NS
EPC = E // (NC * NS)
BLK = 2000
CHUNK = 80
NCH = BLK // CHUNK
RPT = 2 * NP // NS
ZR = 256

_SC_PARAMS = pltpu.CompilerParams(use_tc_tiling_on_sc=False)


def _mesh():
    return plsc.VectorSubcoreMesh(core_axis_name="c", subcore_axis_name="s",
                                  num_cores=NC, num_subcores=NS)


def _make_sc_agg():
    scratch = [
        pltpu.VMEM_SHARED((2 * NP, HH), jnp.float32),
        pltpu.VMEM((BLK,), jnp.int32),
        pltpu.VMEM((BLK,), jnp.int32),
        pltpu.VMEM((BLK,), jnp.int32),
        pltpu.VMEM((NCH, CHUNK), jnp.int32),
        pltpu.VMEM((NCH, CHUNK), jnp.int32),
        pltpu.VMEM((CHUNK, HH), jnp.float32),
        pltpu.VMEM((CHUNK, HH), jnp.float32),
        pltpu.VMEM((CHUNK, HH), jnp.float32),
        pltpu.VMEM((CHUNK, HH), jnp.float32),
        pltpu.VMEM((ZR, HH), jnp.float32),
    ] + [pltpu.SemaphoreType.DMA] * 8

    def body(tbl, src, dst, ety, agg_out, acc_sh, srcb, dstb, etyb,
             gidx, sidx, g0, g1, g2, g3, zbuf, *sems):
        c = lax.axis_index("c")
        s = lax.axis_index("s")
        zv = jnp.zeros((16,), jnp.float32)

        def zrow(i, _):
            for k in range(HH // 16):
                zbuf[i, pl.ds(k * 16, 16)] = zv
            return 0
        lax.fori_loop(0, ZR, zrow, 0)

        rbase = s * RPT

        def zacc(b, _):
            pltpu.sync_copy(zbuf, acc_sh.at[pl.ds(rbase + b * ZR, ZR)])
            return 0
        lax.fori_loop(0, RPT // ZR, zacc, 0)

        plsc.subcore_barrier()

        ebase = s * EPT
        coff = c * N

        def blk_body(b, _):
            eb = ebase + b * BLK
            pltpu.sync_copy(src.at[pl.ds(eb, BLK)], srcb)
            pltpu.sync_copy(dst.at[pl.ds(eb, BLK)], dstb)
            pltpu.sync_copy(ety.at[pl.ds(eb, BLK)], etyb)

            def idx_body(j, _):
                for v in range(CHUNK // 16):
                    sl = pl.ds(j * CHUNK + v * 16, 16)
                    vd = pl.ds(v * 16, 16)
                    gidx[j, vd] = srcb[sl] + coff
                    sidx[j, vd] = etyb[sl] * NP + dstb[sl]
                return 0
            lax.fori_loop(0, NCH, idx_body, 0)

            bufs = (g0, g1, g2, g3)
            gsems, ssems = sems[:4], sems[4:]
            NB, LA = 4, 3
            gd, sd = {}, {}

            def start_gather(j):
                gd[j] = pltpu.async_copy(tbl.at[gidx.at[j]], bufs[j % NB],
                                         gsems[j % NB])

            for k in range(LA):
                start_gather(k)
            for j in range(NCH):
                if j + LA < NCH:
                    if j + LA - NB >= 0:
                        sd.pop(j + LA - NB).wait()
                    start_gather(j + LA)
                gd.pop(j).wait()
                sd[j] = pltpu.async_copy(bufs[j % NB],
                                         acc_sh.at[sidx.at[j]],
                                         ssems[j % NB], add=True)
            for j in sorted(sd):
                sd.pop(j).wait()
            return 0
        lax.fori_loop(0, EPT // BLK, blk_body, 0)

        plsc.subcore_barrier()

        pltpu.sync_copy(acc_sh.at[pl.ds(rbase, RPT)],
                        agg_out.at[c].at[pl.ds(rbase, RPT)])

    return pl.kernel(
        body, out_type=jax.ShapeDtypeStruct((NC, 2 * NP, HH), jnp.float32),
        mesh=_mesh(), scratch_types=scratch, compiler_params=_SC_PARAMS)


def _make_sc_counts():
    scratch = [
        pltpu.VMEM_SHARED((2 * NP, 16), jnp.float32),
        pltpu.VMEM((BLK,), jnp.int32),
        pltpu.VMEM((BLK,), jnp.int32),
        pltpu.VMEM((NCH, CHUNK), jnp.int32),
        pltpu.VMEM((CHUNK, 16), jnp.float32),
        pltpu.VMEM((ZR, 16), jnp.float32),
    ]

    def body(dst, ety, cnt_out, cnt_sh, dstb, etyb, sidx, ones, zcnt):
        c = lax.axis_index("c")
        s = lax.axis_index("s")
        zv = jnp.zeros((16,), jnp.float32)
        ov = jnp.ones((16,), jnp.float32)

        def zc(i, _):
            zcnt[i, pl.ds(0, 16)] = zv
            return 0
        lax.fori_loop(0, ZR, zc, 0)

        def oc(i, _):
            ones[i, pl.ds(0, 16)] = ov
            return 0
        lax.fori_loop(0, CHUNK, oc, 0)

        rbase = s * RPT

        def zcs(b, _):
            pltpu.sync_copy(zcnt, cnt_sh.at[pl.ds(rbase + b * ZR, ZR)])
            return 0
        lax.fori_loop(0, RPT // ZR, zcs, 0)

        plsc.subcore_barrier()

        ebase = (c * NS + s) * EPC

        def blk_body(b, _):
            eb = ebase + b * BLK
            pltpu.sync_copy(dst.at[pl.ds(eb, BLK)], dstb)
            pltpu.sync_copy(ety.at[pl.ds(eb, BLK)], etyb)

            def idx_body(j, _):
                for v in range(CHUNK // 16):
                    sl = pl.ds(j * CHUNK + v * 16, 16)
                    sidx[j, pl.ds(v * 16, 16)] = etyb[sl] * NP + dstb[sl]
                return 0
            lax.fori_loop(0, NCH, idx_body, 0)

            for j in range(NCH):
                pltpu.sync_copy(ones, cnt_sh.at[sidx.at[j]], add=True)
            return 0
        lax.fori_loop(0, EPC // BLK, blk_body, 0)

        plsc.subcore_barrier()

        pltpu.sync_copy(cnt_sh.at[pl.ds(rbase, RPT)],
                        cnt_out.at[c].at[pl.ds(rbase, RPT)])

    return pl.kernel(
        body, out_type=jax.ShapeDtypeStruct((NC, 2 * NP, 16), jnp.float32),
        mesh=_mesh(), scratch_types=scratch, compiler_params=_SC_PARAMS)


def _tc_input_proj(feature, W_in, b_in2):
    B = 1000

    def body(f_ref, w_ref, b_ref, x_ref, xh_ref):
        act = jnp.dot(f_ref[...], w_ref[...],
                      preferred_element_type=jnp.float32) + b_ref[...]
        act = jnp.where(act >= 0, act, 0.01 * act)
        x_ref[...] = act
        xh_ref[0] = act[:, :HH]
        xh_ref[1] = act[:, HH:]

    return pl.pallas_call(
        body,
        grid=(N // B,),
        in_specs=[
            pl.BlockSpec((B, 16), lambda i: (i, 0)),
            pl.BlockSpec((16, DH), lambda i: (0, 0)),
            pl.BlockSpec((1, DH), lambda i: (0, 0)),
        ],
        out_specs=[
            pl.BlockSpec((B, DH), lambda i: (i, 0)),
            pl.BlockSpec((2, B, HH), lambda i: (0, i, 0)),
        ],
        out_shape=[
            jax.ShapeDtypeStruct((N, DH), jnp.float32),
            jax.ShapeDtypeStruct((2, N, HH), jnp.float32),
        ],
    )(feature, W_in, b_in2)


def _tc_combine(x, agg4, cnt4, rel_w, root_w, bias2, final, W_out=None,
                b_out2=None):
    B = 1000

    def body(x_ref, a_ref, c_ref, rw_ref, rt_ref, b_ref, *refs):
        if final:
            wo_ref, bo_ref = refs[0], refs[1]
            refs = refs[2:]
        acc = jnp.dot(x_ref[...], rt_ref[...],
                      preferred_element_type=jnp.float32) + b_ref[...]
        for r in range(NR):
            cnt = c_ref[0, r, :, 0:1] + c_ref[1, r, :, 0:1]
            inv = 1.0 / jnp.maximum(cnt, 1.0)
            for h in range(2):
                m = a_ref[h, r] * inv
                acc += jnp.dot(m, rw_ref[r, h * HH:(h + 1) * HH, :],
                               preferred_element_type=jnp.float32)
        if final:
            o = jnp.dot(acc, wo_ref[...],
                        preferred_element_type=jnp.float32) + bo_ref[...]
            refs[0][...] = o
        else:
            refs[0][...] = acc
            refs[1][0] = acc[:, :HH]
            refs[1][1] = acc[:, HH:]

    in_specs = [
        pl.BlockSpec((B, DH), lambda i: (i, 0)),
        pl.BlockSpec((NC, NR, B, HH), lambda i: (0, 0, i, 0)),
        pl.BlockSpec((NC, NR, B, 16), lambda i: (0, 0, i, 0)),
        pl.BlockSpec((NR, DH, DH), lambda i: (0, 0, 0)),
        pl.BlockSpec((DH, DH), lambda i: (0, 0)),
        pl.BlockSpec((1, DH), lambda i: (0, 0)),
    ]
    args = [x, agg4, cnt4, rel_w, root_w, bias2]
    if final:
        in_specs += [
            pl.BlockSpec((DH, 3), lambda i: (0, 0)),
            pl.BlockSpec((1, 3), lambda i: (0, 0)),
        ]
        args += [W_out, b_out2]
        out_specs = pl.BlockSpec((B, 3), lambda i: (i, 0))
        out_shape = jax.ShapeDtypeStruct((N, 3), jnp.float32)
    else:
        out_specs = [
            pl.BlockSpec((B, DH), lambda i: (i, 0)),
            pl.BlockSpec((2, B, HH), lambda i: (0, i, 0)),
        ]
        out_shape = [
            jax.ShapeDtypeStruct((N, DH), jnp.float32),
            jax.ShapeDtypeStruct((2, N, HH), jnp.float32),
        ]

    return pl.pallas_call(body, grid=(N // B,), in_specs=in_specs,
                          out_specs=out_specs, out_shape=out_shape)(*args)


def kernel(feature, index, consisenty, W_in, b_in, rel_weight, root_w,
           rgcn_bias, W_out, b_out):
    src = index[0].astype(jnp.int32)
    dst = index[1].astype(jnp.int32)
    ety = consisenty.astype(jnp.int32)

    x0, xh0 = _tc_input_proj(feature.astype(jnp.float32), W_in,
                             b_in.reshape(1, DH))

    sc_agg = _make_sc_agg()
    sc_counts = _make_sc_counts()

    cnt = sc_counts(dst, ety)
    cnt4 = cnt.reshape(NC, NR, NP, 16)

    agg1 = sc_agg(xh0.reshape(2 * N, HH), src, dst, ety)

    x1, xh1 = _tc_combine(x0, agg1.reshape(NC, NR, NP, HH), cnt4,
                          rel_weight, root_w, rgcn_bias.reshape(1, DH),
                          final=False)

    agg2 = sc_agg(xh1.reshape(2 * N, HH), src, dst, ety)

    out = _tc_combine(x1, agg2.reshape(NC, NR, NP, HH), cnt4,
                      rel_weight, root_w, rgcn_bias.reshape(1, DH),
                      final=True, W_out=W_out, b_out2=b_out.reshape(1, 3))
    return out


# speedup vs baseline: 13.7082x; 1.0424x over previous
import jax
import jax.numpy as jnp
from jax import lax
from jax.experimental import pallas as pl
from jax.experimental.pallas import tpu as pltpu
from jax.experimental.pallas import tpu_sc as plsc

N = 10000
NP = 10240
E = 320000
DH = 128
HH = 64
NR = 2
NC = 2
NS = 16
EPT = E // NS
EPC = E // (NC * NS)
BLK = 2000
CHUNK = 80
NCH = BLK // CHUNK
RPT = 2 * NP // NS
TPR = NS // NR
ZR = 256

_SC_PARAMS = pltpu.CompilerParams(use_tc_tiling_on_sc=False)


def _mesh():
    return plsc.VectorSubcoreMesh(core_axis_name="c", subcore_axis_name="s",
                                  num_cores=NC, num_subcores=NS)


def _make_sc_agg():
    scratch = [
        pltpu.VMEM_SHARED((2 * NP, HH), jnp.float32),
        pltpu.VMEM((BLK,), jnp.int32),
        pltpu.VMEM((BLK,), jnp.int32),
        pltpu.VMEM((BLK,), jnp.int32),
        pltpu.VMEM((NCH, CHUNK), jnp.int32),
        pltpu.VMEM((NCH, CHUNK), jnp.int32),
        pltpu.VMEM((CHUNK, HH), jnp.float32),
        pltpu.VMEM((CHUNK, HH), jnp.float32),
        pltpu.VMEM((CHUNK, HH), jnp.float32),
        pltpu.VMEM((CHUNK, HH), jnp.float32),
        pltpu.VMEM((ZR, HH), jnp.float32),
    ] + [pltpu.SemaphoreType.DMA] * 8

    def body(tbl, idx2, ety, agg_out, acc_sh, srcb, dstb, etyb,
             gidx, sidx, g0, g1, g2, g3, zbuf, *sems):
        c = lax.axis_index("c")
        s = lax.axis_index("s")
        zv = jnp.zeros((16,), jnp.float32)

        def zrow(i, _):
            for k in range(HH // 16):
                zbuf[i, pl.ds(k * 16, 16)] = zv
            return 0
        lax.fori_loop(0, ZR, zrow, 0)

        rbase = s * RPT

        def zacc(b, _):
            pltpu.sync_copy(zbuf, acc_sh.at[pl.ds(rbase + b * ZR, ZR)])
            return 0
        lax.fori_loop(0, RPT // ZR, zacc, 0)

        plsc.subcore_barrier()

        ebase = s * EPT
        mytbl = tbl.at[c]

        def blk_body(b, _):
            eb = ebase + b * BLK
            pltpu.sync_copy(idx2.at[0].at[pl.ds(eb, BLK)], srcb)
            pltpu.sync_copy(idx2.at[1].at[pl.ds(eb, BLK)], dstb)
            pltpu.sync_copy(ety.at[pl.ds(eb, BLK)], etyb)

            def idx_body(j, _):
                for v in range(CHUNK // 16):
                    sl = pl.ds(j * CHUNK + v * 16, 16)
                    vd = pl.ds(v * 16, 16)
                    gidx[j, vd] = srcb[sl]
                    sidx[j, vd] = etyb[sl] * NP + dstb[sl]
                return 0
            lax.fori_loop(0, NCH, idx_body, 0)

            bufs = (g0, g1, g2, g3)
            gsems, ssems = sems[:4], sems[4:]
            NB, LA = 4, 3
            gd, sd = {}, {}

            def start_gather(j):
                gd[j] = pltpu.async_copy(mytbl.at[gidx.at[j]], bufs[j % NB],
                                         gsems[j % NB])

            for k in range(LA):
                start_gather(k)
            for j in range(NCH):
                if j + LA < NCH:
                    if j + LA - NB >= 0:
                        sd.pop(j + LA - NB).wait()
                    start_gather(j + LA)
                gd.pop(j).wait()
                sd[j] = pltpu.async_copy(bufs[j % NB],
                                         acc_sh.at[sidx.at[j]],
                                         ssems[j % NB], add=True)
            for j in sorted(sd):
                sd.pop(j).wait()
            return 0
        lax.fori_loop(0, EPT // BLK, blk_body, 0)

        plsc.subcore_barrier()

        r = s // TPR
        lbase = (s - r * TPR) * RPT
        pltpu.sync_copy(acc_sh.at[pl.ds(rbase, RPT)],
                        agg_out.at[c].at[r].at[pl.ds(lbase, RPT)])

    return pl.kernel(
        body,
        out_type=jax.ShapeDtypeStruct((NC, NR, NP, HH), jnp.float32),
        mesh=_mesh(), scratch_types=scratch, compiler_params=_SC_PARAMS)


def _make_sc_counts():
    scratch = [
        pltpu.VMEM_SHARED((2 * NP, 16), jnp.float32),
        pltpu.VMEM((BLK,), jnp.int32),
        pltpu.VMEM((BLK,), jnp.int32),
        pltpu.VMEM((NCH, CHUNK), jnp.int32),
        pltpu.VMEM((CHUNK, 16), jnp.float32),
        pltpu.VMEM((ZR, 16), jnp.float32),
    ] + [pltpu.SemaphoreType.DMA] * 4

    def body(idx2, ety, cnt_out, cnt_sh, dstb, etyb, sidx, ones, zcnt,
             *ssems):
        c = lax.axis_index("c")
        s = lax.axis_index("s")
        zv = jnp.zeros((16,), jnp.float32)
        ov = jnp.ones((16,), jnp.float32)

        def zc(i, _):
            zcnt[i, pl.ds(0, 16)] = zv
            return 0
        lax.fori_loop(0, ZR, zc, 0)

        def oc(i, _):
            ones[i, pl.ds(0, 16)] = ov
            return 0
        lax.fori_loop(0, CHUNK, oc, 0)

        rbase = s * RPT

        def zcs(b, _):
            pltpu.sync_copy(zcnt, cnt_sh.at[pl.ds(rbase + b * ZR, ZR)])
            return 0
        lax.fori_loop(0, RPT // ZR, zcs, 0)

        plsc.subcore_barrier()

        ebase = (c * NS + s) * EPC

        def blk_body(b, _):
            eb = ebase + b * BLK
            pltpu.sync_copy(idx2.at[1].at[pl.ds(eb, BLK)], dstb)
            pltpu.sync_copy(ety.at[pl.ds(eb, BLK)], etyb)

            def idx_body(j, _):
                for v in range(CHUNK // 16):
                    sl = pl.ds(j * CHUNK + v * 16, 16)
                    sidx[j, pl.ds(v * 16, 16)] = etyb[sl] * NP + dstb[sl]
                return 0
            lax.fori_loop(0, NCH, idx_body, 0)

            sd = {}
            for j in range(NCH):
                if j - 4 >= 0:
                    sd.pop(j - 4).wait()
                sd[j] = pltpu.async_copy(ones, cnt_sh.at[sidx.at[j]],
                                         ssems[j % 4], add=True)
            for j in sorted(sd):
                sd.pop(j).wait()
            return 0
        lax.fori_loop(0, EPC // BLK, blk_body, 0)

        plsc.subcore_barrier()

        r = s // TPR
        lbase = (s - r * TPR) * RPT
        pltpu.sync_copy(cnt_sh.at[pl.ds(rbase, RPT)],
                        cnt_out.at[c].at[r].at[pl.ds(lbase, RPT)])

    return pl.kernel(
        body,
        out_type=jax.ShapeDtypeStruct((NC, NR, NP, 16), jnp.float32),
        mesh=_mesh(), scratch_types=scratch, compiler_params=_SC_PARAMS)


def _tc_input_proj(feature, W_in, b_in2):
    B = 1000

    def body(f_ref, w_ref, b_ref, x_ref, xh_ref):
        act = jnp.dot(f_ref[...], w_ref[...],
                      preferred_element_type=jnp.float32) + b_ref[...]
        act = jnp.where(act >= 0, act, 0.01 * act)
        x_ref[...] = act
        xh_ref[0] = act[:, :HH]
        xh_ref[1] = act[:, HH:]

    return pl.pallas_call(
        body,
        grid=(N // B,),
        in_specs=[
            pl.BlockSpec((B, 16), lambda i: (i, 0)),
            pl.BlockSpec((16, DH), lambda i: (0, 0)),
            pl.BlockSpec((1, DH), lambda i: (0, 0)),
        ],
        out_specs=[
            pl.BlockSpec((B, DH), lambda i: (i, 0)),
            pl.BlockSpec((2, B, HH), lambda i: (0, i, 0)),
        ],
        out_shape=[
            jax.ShapeDtypeStruct((N, DH), jnp.float32),
            jax.ShapeDtypeStruct((2, N, HH), jnp.float32),
        ],
    )(feature, W_in, b_in2)


def _tc_combine(x, agg4, cnt4, rel_w, root_w, bias2, final, W_out=None,
                b_out2=None):
    B = 1000

    def body(x_ref, a_ref, c_ref, rw_ref, rt_ref, b_ref, *refs):
        if final:
            wo_ref, bo_ref = refs[0], refs[1]
            refs = refs[2:]
        acc = jnp.dot(x_ref[...], rt_ref[...],
                      preferred_element_type=jnp.float32) + b_ref[...]
        for r in range(NR):
            cnt = c_ref[0, r, :, 0:1] + c_ref[1, r, :, 0:1]
            inv = 1.0 / jnp.maximum(cnt, 1.0)
            for h in range(2):
                m = a_ref[h, r] * inv
                acc += jnp.dot(m, rw_ref[r, h * HH:(h + 1) * HH, :],
                               preferred_element_type=jnp.float32)
        if final:
            o = jnp.dot(acc, wo_ref[...],
                        preferred_element_type=jnp.float32) + bo_ref[...]
            refs[0][...] = o
        else:
            refs[0][...] = acc
            refs[1][0] = acc[:, :HH]
            refs[1][1] = acc[:, HH:]

    in_specs = [
        pl.BlockSpec((B, DH), lambda i: (i, 0)),
        pl.BlockSpec((NC, NR, B, HH), lambda i: (0, 0, i, 0)),
        pl.BlockSpec((NC, NR, B, 16), lambda i: (0, 0, i, 0)),
        pl.BlockSpec((NR, DH, DH), lambda i: (0, 0, 0)),
        pl.BlockSpec((DH, DH), lambda i: (0, 0)),
        pl.BlockSpec((1, DH), lambda i: (0, 0)),
    ]
    args = [x, agg4, cnt4, rel_w, root_w, bias2]
    if final:
        in_specs += [
            pl.BlockSpec((DH, 3), lambda i: (0, 0)),
            pl.BlockSpec((1, 3), lambda i: (0, 0)),
        ]
        args += [W_out, b_out2]
        out_specs = pl.BlockSpec((B, 3), lambda i: (i, 0))
        out_shape = jax.ShapeDtypeStruct((N, 3), jnp.float32)
    else:
        out_specs = [
            pl.BlockSpec((B, DH), lambda i: (i, 0)),
            pl.BlockSpec((2, B, HH), lambda i: (0, i, 0)),
        ]
        out_shape = [
            jax.ShapeDtypeStruct((N, DH), jnp.float32),
            jax.ShapeDtypeStruct((2, N, HH), jnp.float32),
        ]

    return pl.pallas_call(body, grid=(N // B,), in_specs=in_specs,
                          out_specs=out_specs, out_shape=out_shape)(*args)


def kernel(feature, index, consisenty, W_in, b_in, rel_weight, root_w,
           rgcn_bias, W_out, b_out):
    idx2 = index.astype(jnp.int32)
    ety = consisenty.astype(jnp.int32)

    x0, xh0 = _tc_input_proj(feature.astype(jnp.float32), W_in,
                             b_in.reshape(1, DH))

    sc_agg = _make_sc_agg()
    sc_counts = _make_sc_counts()

    cnt4 = sc_counts(idx2, ety)
    agg1 = sc_agg(xh0, idx2, ety)

    x1, xh1 = _tc_combine(x0, agg1, cnt4, rel_weight, root_w,
                          rgcn_bias.reshape(1, DH), final=False)

    agg2 = sc_agg(xh1, idx2, ety)

    out = _tc_combine(x1, agg2, cnt4, rel_weight, root_w,
                      rgcn_bias.reshape(1, DH), final=True, W_out=W_out,
                      b_out2=b_out.reshape(1, 3))
    return out
